# Initial kernel scaffold; baseline (speedup 1.0000x reference)
import jax
import jax.numpy as jnp
from jax.experimental import pallas as pl


def kernel(z, y):
    raise NotImplementedError("write your pallas kernel here")


# speedup vs baseline: 1.5630x; 1.5630x over previous
import functools

import jax
import jax.numpy as jnp
from jax.experimental import pallas as pl

_S = 5.0
_N = 4096
_B = 512


def _body(zc_ref, yc_ref, zr_ref, yr_ref, sum_ref, cnt_ref):
    bi = pl.program_id(0)
    bj = pl.program_id(1)

    @pl.when(jnp.logical_and(bi == 0, bj == 0))
    def _init():
        sum_ref[...] = jnp.zeros((1, 1), jnp.float32)
        cnt_ref[...] = jnp.zeros((1, 1), jnp.float32)

    def compute(diagonal):
        zi = zc_ref[...]
        yi = yc_ref[...]
        zj = zr_ref[...]
        yj = yr_ref[...]
        dy = yi - yj
        mask = dy != 0.0
        if diagonal:
            row = jax.lax.broadcasted_iota(jnp.int32, (_B, _B), 0)
            col = jax.lax.broadcasted_iota(jnp.int32, (_B, _B), 1)
            mask = jnp.logical_and(mask, col > row)
        a = (jnp.sign(dy) * (-_S)) * (zi - zj)
        vals = jnp.log1p(jnp.exp(a))
        sum_ref[...] += jnp.sum(jnp.where(mask, vals, 0.0), keepdims=True)
        cnt_ref[...] += jnp.sum(mask.astype(jnp.float32), keepdims=True)

    @pl.when(bj > bi)
    def _full():
        compute(False)

    @pl.when(bj == bi)
    def _diag():
        compute(True)


@jax.jit
def kernel(z, y):
    z = z.reshape(-1)
    y = y.reshape(-1)
    nb = _N // _B
    grid = (nb, nb)
    sum_half, cnt_half = pl.pallas_call(
        _body,
        grid=grid,
        in_specs=[
            pl.BlockSpec((_B, 1), lambda i, j: (i, 0)),
            pl.BlockSpec((_B, 1), lambda i, j: (i, 0)),
            pl.BlockSpec((1, _B), lambda i, j: (0, j)),
            pl.BlockSpec((1, _B), lambda i, j: (0, j)),
        ],
        out_specs=[
            pl.BlockSpec((1, 1), lambda i, j: (0, 0)),
            pl.BlockSpec((1, 1), lambda i, j: (0, 0)),
        ],
        out_shape=[
            jax.ShapeDtypeStruct((1, 1), jnp.float32),
            jax.ShapeDtypeStruct((1, 1), jnp.float32),
        ],
    )(
        z.reshape(_N, 1),
        y.reshape(_N, 1),
        z.reshape(1, _N),
        y.reshape(1, _N),
    )
    s = sum_half[0, 0]
    c = cnt_half[0, 0]
    return jnp.where(c > 0, s / jnp.maximum(c, 1.0), 0.0)


# speedup vs baseline: 2.0831x; 1.3328x over previous
import jax
import jax.numpy as jnp
from jax import lax
from jax.experimental import pallas as pl
from jax.experimental.pallas import tpu as pltpu

_S = 5.0
_LOG2E = 1.4426950408889634
_N = 4096
_B = 512
_NB = _N // _B
_SIGNBIT = 0x80000000


def _body(blk_ref, zc_ref, yc_ref, zr_ref, yr_ref, sum_ref, cnt_ref):
    k = pl.program_id(0)
    bi = blk_ref[0, k]
    bj = blk_ref[1, k]

    @pl.when(k == 0)
    def _init():
        sum_ref[...] = jnp.zeros((1, 1), jnp.float32)
        cnt_ref[...] = jnp.zeros((1, 1), jnp.float32)

    alpha = jnp.float32(_S * _LOG2E)
    szi = zc_ref[...] * alpha
    szj = zr_ref[...] * alpha
    dy = yc_ref[...] - yr_ref[...]
    dz = szj - szi
    sbit = lax.bitcast_convert_type(dy, jnp.uint32) & jnp.uint32(_SIGNBIT)
    a = lax.bitcast_convert_type(lax.bitcast_convert_type(dz, jnp.uint32) ^ sbit, jnp.float32)
    vals = jnp.log1p(jnp.exp2(a))
    mask = dy != 0.0
    bs = jnp.sum(jnp.where(mask, vals, 0.0), keepdims=True)
    bc = jnp.sum(jnp.where(mask, 1.0, 0.0), keepdims=True)
    w = jnp.where(bi == bj, 0.5, 1.0)
    sum_ref[...] += w * bs
    cnt_ref[...] += w * bc


@jax.jit
def kernel(z, y):
    z = z.reshape(-1)
    y = y.reshape(-1)
    blocks = [(i, j) for i in range(_NB) for j in range(i, _NB)]
    blk = jnp.asarray([[b[0] for b in blocks], [b[1] for b in blocks]], dtype=jnp.int32)
    nsteps = len(blocks)

    grid_spec = pltpu.PrefetchScalarGridSpec(
        num_scalar_prefetch=1,
        grid=(nsteps,),
        in_specs=[
            pl.BlockSpec((_B, 1), lambda k, blk: (blk[0, k], 0)),
            pl.BlockSpec((_B, 1), lambda k, blk: (blk[0, k], 0)),
            pl.BlockSpec((1, _B), lambda k, blk: (0, blk[1, k])),
            pl.BlockSpec((1, _B), lambda k, blk: (0, blk[1, k])),
        ],
        out_specs=[
            pl.BlockSpec((1, 1), lambda k, blk: (0, 0)),
            pl.BlockSpec((1, 1), lambda k, blk: (0, 0)),
        ],
    )
    sum_half, cnt_half = pl.pallas_call(
        _body,
        grid_spec=grid_spec,
        out_shape=[
            jax.ShapeDtypeStruct((1, 1), jnp.float32),
            jax.ShapeDtypeStruct((1, 1), jnp.float32),
        ],
    )(
        blk,
        z.reshape(_N, 1),
        y.reshape(_N, 1),
        z.reshape(1, _N),
        y.reshape(1, _N),
    )
    s = sum_half[0, 0]
    c = cnt_half[0, 0]
    return jnp.where(c > 0, s / jnp.maximum(c, 1.0), 0.0)


# speedup vs baseline: 2.3862x; 1.1455x over previous
import jax
import jax.numpy as jnp
from jax import lax
from jax.experimental import pallas as pl
from jax.experimental.pallas import tpu as pltpu

_S = 5.0
_LOG2E = 1.4426950408889634
_N = 4096
_B = 512
_NB = _N // _B
_SIGNBIT = 0x80000000


def _body(blk_ref, zc_ref, yc_ref, zr_ref, yr_ref, sum_ref, cnt_ref):
    k = pl.program_id(0)
    bi = blk_ref[0, k]
    bj = blk_ref[1, k]

    @pl.when(k == 0)
    def _init():
        sum_ref[...] = jnp.zeros((1, 1), jnp.float32)
        cnt_ref[...] = jnp.zeros((1, 1), jnp.float32)

    alpha = jnp.float32(_S * _LOG2E)
    szi = zc_ref[...] * alpha
    szj = zr_ref[...] * alpha
    dy = yc_ref[...] - yr_ref[...]
    dz = szj - szi
    sbit = lax.bitcast_convert_type(dy, jnp.uint32) & jnp.uint32(_SIGNBIT)
    a = lax.bitcast_convert_type(lax.bitcast_convert_type(dz, jnp.uint32) ^ sbit, jnp.float32)
    vals = jnp.log(1.0 + jnp.exp2(a))
    mask = dy != 0.0
    bs = jnp.sum(jnp.where(mask, vals, 0.0), keepdims=True)
    bc = jnp.sum(jnp.where(mask, 1.0, 0.0), keepdims=True)
    w = jnp.where(bi == bj, 0.5, 1.0)
    sum_ref[...] += w * bs
    cnt_ref[...] += w * bc


@jax.jit
def kernel(z, y):
    z = z.reshape(-1)
    y = y.reshape(-1)
    blocks = [(i, j) for i in range(_NB) for j in range(i, _NB)]
    blk = jnp.asarray([[b[0] for b in blocks], [b[1] for b in blocks]], dtype=jnp.int32)
    nsteps = len(blocks)

    grid_spec = pltpu.PrefetchScalarGridSpec(
        num_scalar_prefetch=1,
        grid=(nsteps,),
        in_specs=[
            pl.BlockSpec((_B, 1), lambda k, blk: (blk[0, k], 0)),
            pl.BlockSpec((_B, 1), lambda k, blk: (blk[0, k], 0)),
            pl.BlockSpec((1, _B), lambda k, blk: (0, blk[1, k])),
            pl.BlockSpec((1, _B), lambda k, blk: (0, blk[1, k])),
        ],
        out_specs=[
            pl.BlockSpec((1, 1), lambda k, blk: (0, 0)),
            pl.BlockSpec((1, 1), lambda k, blk: (0, 0)),
        ],
    )
    sum_half, cnt_half = pl.pallas_call(
        _body,
        grid_spec=grid_spec,
        out_shape=[
            jax.ShapeDtypeStruct((1, 1), jnp.float32),
            jax.ShapeDtypeStruct((1, 1), jnp.float32),
        ],
    )(
        blk,
        z.reshape(_N, 1),
        y.reshape(_N, 1),
        z.reshape(1, _N),
        y.reshape(1, _N),
    )
    s = sum_half[0, 0]
    c = cnt_half[0, 0]
    return jnp.where(c > 0, s / jnp.maximum(c, 1.0), 0.0)


# speedup vs baseline: 2.9806x; 1.2491x over previous
import jax
import jax.numpy as jnp
from jax import lax
from jax.experimental import pallas as pl
from jax.experimental.pallas import tpu as pltpu

_S = 5.0
_LOG2E = 1.4426950408889634
_N = 4096
_B = 1024
_NB = _N // _B
_SIGNBIT = 0x80000000


def _body(blk_ref, zc_ref, yc_ref, zr_ref, yr_ref, sum_ref, cnt_ref):
    k = pl.program_id(0)
    bi = blk_ref[0, k]
    bj = blk_ref[1, k]

    @pl.when(k == 0)
    def _init():
        sum_ref[...] = jnp.zeros((1, 1), jnp.float32)
        cnt_ref[...] = jnp.zeros((1, 1), jnp.float32)

    alpha = jnp.float32(_S * _LOG2E)
    szi = zc_ref[...] * alpha
    szj = zr_ref[...] * alpha
    dy = yc_ref[...] - yr_ref[...]
    dz = szj - szi
    sbit = lax.bitcast_convert_type(dy, jnp.uint32) & jnp.uint32(_SIGNBIT)
    a = lax.bitcast_convert_type(lax.bitcast_convert_type(dz, jnp.uint32) ^ sbit, jnp.float32)
    vals = jnp.log(1.0 + jnp.exp2(a))
    mask = dy != 0.0
    bs = jnp.sum(jnp.where(mask, vals, 0.0), keepdims=True)
    bc = jnp.sum(jnp.where(mask, 1.0, 0.0), keepdims=True)
    w = jnp.where(bi == bj, 0.5, 1.0)
    sum_ref[...] += w * bs
    cnt_ref[...] += w * bc


@jax.jit
def kernel(z, y):
    z = z.reshape(-1)
    y = y.reshape(-1)
    blocks = [(i, j) for i in range(_NB) for j in range(i, _NB)]
    blk = jnp.asarray([[b[0] for b in blocks], [b[1] for b in blocks]], dtype=jnp.int32)
    nsteps = len(blocks)

    grid_spec = pltpu.PrefetchScalarGridSpec(
        num_scalar_prefetch=1,
        grid=(nsteps,),
        in_specs=[
            pl.BlockSpec((_B, 1), lambda k, blk: (blk[0, k], 0)),
            pl.BlockSpec((_B, 1), lambda k, blk: (blk[0, k], 0)),
            pl.BlockSpec((1, _B), lambda k, blk: (0, blk[1, k])),
            pl.BlockSpec((1, _B), lambda k, blk: (0, blk[1, k])),
        ],
        out_specs=[
            pl.BlockSpec((1, 1), lambda k, blk: (0, 0)),
            pl.BlockSpec((1, 1), lambda k, blk: (0, 0)),
        ],
    )
    sum_half, cnt_half = pl.pallas_call(
        _body,
        grid_spec=grid_spec,
        out_shape=[
            jax.ShapeDtypeStruct((1, 1), jnp.float32),
            jax.ShapeDtypeStruct((1, 1), jnp.float32),
        ],
    )(
        blk,
        z.reshape(_N, 1),
        y.reshape(_N, 1),
        z.reshape(1, _N),
        y.reshape(1, _N),
    )
    s = sum_half[0, 0]
    c = cnt_half[0, 0]
    return jnp.where(c > 0, s / jnp.maximum(c, 1.0), 0.0)
